# Initial kernel scaffold; baseline (speedup 1.0000x reference)
import functools

import jax
import jax.numpy as jnp
from jax import lax
from jax.experimental import pallas as pl
from jax.experimental.pallas import tpu as pltpu
from jax.experimental.pallas import tpu_sc as plsc

N = 10000
F = 128
E = 320000
NC = 2
NS = 16
NW = NC * NS
CH = 125
PER_TILE = E // NW
STEPS = PER_TILE // CH
ROWS_PER_SUB = N // NS
INIT_STEPS = ROWS_PER_SUB // CH

_mesh = plsc.VectorSubcoreMesh(core_axis_name="c", subcore_axis_name="s")


def _agg_body(h_hbm, src_hbm, dst_hbm, out_hbm,
              src_v, dst_v, rows0, rows1, cbuf, acc, sem0, sem1):
  c = lax.axis_index("c")
  s = lax.axis_index("s")
  widx = c * NS + s

  row0 = s * ROWS_PER_SUB
  for t in range(INIT_STEPS):
    r = row0 + t * CH
    pltpu.sync_copy(h_hbm.at[pl.ds(r, CH)], cbuf)
    pltpu.sync_copy(cbuf, acc.at[pl.ds(r, CH)])
  plsc.subcore_barrier()

  pltpu.sync_copy(src_hbm.at[pl.ds(widx * STEPS, STEPS)], src_v)
  pltpu.sync_copy(dst_hbm.at[pl.ds(widx * STEPS, STEPS)], dst_v)

  pltpu.async_copy(h_hbm.at[src_v.at[0]], rows0, sem0)

  def step(j, carry):
    @pl.when(j + 1 < STEPS)
    def _():
      pltpu.async_copy(h_hbm.at[src_v.at[j + 1]], rows1, sem1)
    pltpu.make_async_copy(h_hbm.at[src_v.at[j]], rows0, sem0).wait()
    pltpu.sync_copy(rows0, acc.at[dst_v.at[j]], add=True)

    @pl.when(j + 2 < STEPS)
    def _():
      pltpu.async_copy(h_hbm.at[src_v.at[j + 2]], rows0, sem0)
    pltpu.make_async_copy(h_hbm.at[src_v.at[j + 1]], rows1, sem1).wait()
    pltpu.sync_copy(rows1, acc.at[dst_v.at[j + 1]], add=True)
    return carry

  lax.fori_loop(0, STEPS // 2, lambda i, cr: step(i * 2, cr), 0)

  plsc.subcore_barrier()
  for t in range(INIT_STEPS):
    r = row0 + t * CH
    pltpu.sync_copy(acc.at[pl.ds(r, CH)], cbuf)
    pltpu.sync_copy(cbuf, out_hbm.at[c, pl.ds(r, CH)])


_agg = pl.kernel(
    _agg_body,
    out_type=jax.ShapeDtypeStruct((NC, N, F), jnp.float32),
    mesh=_mesh,
    scratch_types=[
        pltpu.VMEM((STEPS, CH), jnp.int32),
        pltpu.VMEM((STEPS, CH), jnp.int32),
        pltpu.VMEM((CH, F), jnp.float32),
        pltpu.VMEM((CH, F), jnp.float32),
        pltpu.VMEM((CH, F), jnp.float32),
        pltpu.VMEM_SHARED((N, F), jnp.float32),
        pltpu.SemaphoreType.DMA,
        pltpu.SemaphoreType.DMA,
    ],
)


def _mlp_bn_body(pA, pB, h, Wa, ba, Wb, bb, g, be, out):
  z = pA[...] + pB[...] - h[...]
  u = jnp.maximum(jnp.dot(z, Wa[...], preferred_element_type=jnp.float32)
                  + ba[...], 0.0)
  v = jnp.dot(u, Wb[...], preferred_element_type=jnp.float32) + bb[...]
  v = jnp.maximum(v, 0.0)
  m = jnp.mean(v, axis=0, keepdims=True)
  var = jnp.mean((v - m) * (v - m), axis=0, keepdims=True)
  out[...] = (v - m) * lax.rsqrt(var + 1e-5) * g[...] + be[...]


def _mlp_bn(parts, h, Wa, ba, Wb, bb, g, be, dout):
  return pl.pallas_call(
      _mlp_bn_body,
      out_shape=jax.ShapeDtypeStruct((N, dout), jnp.float32),
  )(parts[0], parts[1], h, Wa, ba, Wb, bb, g, be)


def kernel(x, edge_index, W1a, b1a, W1b, b1b, g1, be1,
           W2a, b2a, W2b, b2b, g2, be2):
  src2 = edge_index[0].reshape(NW * STEPS, CH)
  dst2 = edge_index[1].reshape(NW * STEPS, CH)
  parts1 = _agg(x, src2, dst2)
  h1 = _mlp_bn(parts1, x, W1a, b1a, W1b, b1b, g1, be1, F)
  parts2 = _agg(h1, src2, dst2)
  h2 = _mlp_bn(parts2, h1, W2a, b2a, W2b, b2b, g2, be2, 2)
  return h2


# speedup vs baseline: 10.0342x; 10.0342x over previous
import functools

import jax
import jax.numpy as jnp
from jax import lax
from jax.experimental import pallas as pl
from jax.experimental.pallas import tpu as pltpu
from jax.experimental.pallas import tpu_sc as plsc

N = 10000
F = 128
E = 320000
NC = 2
NS = 16
NW = NC * NS
CH = 100
PER_TILE = E // NW
STEPS = PER_TILE // CH
ROWS_PER_SUB = N // NS


def _agg_body(h_hbm, src_hbm, dst_hbm, out_hbm,
              src_v, dst_v, rows0, rows1, acc, sem0, sem1):
  c = lax.axis_index("c")
  s = lax.axis_index("s")
  widx = c * NS + s

  row0 = s * ROWS_PER_SUB
  for t in range(6):
    r = row0 + t * CH
    n = CH if t < 6 - 1 else ROWS_PER_SUB - (6 - 1) * CH
    pltpu.sync_copy(h_hbm.at[pl.ds(r, n)], rows0.at[pl.ds(0, n)])
    pltpu.sync_copy(rows0.at[pl.ds(0, n)], acc.at[pl.ds(r, n)])
  plsc.subcore_barrier()

  pltpu.sync_copy(src_hbm.at[pl.ds(widx * STEPS, STEPS)], src_v)
  pltpu.sync_copy(dst_hbm.at[pl.ds(widx * STEPS, STEPS)], dst_v)

  pltpu.async_copy(h_hbm.at[src_v.at[0]], rows0, sem0)

  def step(j, carry):
    @pl.when(j + 1 < STEPS)
    def _():
      pltpu.async_copy(h_hbm.at[src_v.at[j + 1]], rows1, sem1)
    pltpu.make_async_copy(h_hbm.at[src_v.at[j]], rows0, sem0).wait()
    pltpu.sync_copy(rows0, acc.at[dst_v.at[j]], add=True)

    @pl.when(j + 2 < STEPS)
    def _():
      pltpu.async_copy(h_hbm.at[src_v.at[j + 2]], rows0, sem0)
    pltpu.make_async_copy(h_hbm.at[src_v.at[j + 1]], rows1, sem1).wait()
    pltpu.sync_copy(rows1, acc.at[dst_v.at[j + 1]], add=True)
    return carry

  lax.fori_loop(0, STEPS // 2, lambda i, cr: step(i * 2, cr), 0)

  plsc.subcore_barrier()
  for t in range(6):
    r = row0 + t * CH
    n = CH if t < 6 - 1 else ROWS_PER_SUB - (6 - 1) * CH
    pltpu.sync_copy(acc.at[pl.ds(r, n)], rows0.at[pl.ds(0, n)])
    pltpu.sync_copy(rows0.at[pl.ds(0, n)], out_hbm.at[c, pl.ds(r, n)])


_agg = pl.kernel(
    _agg_body,
    out_type=jax.ShapeDtypeStruct((NC, N, F), jnp.float32),
    mesh=plsc.VectorSubcoreMesh(core_axis_name="c", subcore_axis_name="s"),
    scratch_types=[
        pltpu.VMEM((STEPS, CH), jnp.int32),
        pltpu.VMEM((STEPS, CH), jnp.int32),
        pltpu.VMEM((CH, F), jnp.float32),
        pltpu.VMEM((CH, F), jnp.float32),
        pltpu.VMEM_SHARED((N, F), jnp.float32),
        pltpu.SemaphoreType.DMA,
        pltpu.SemaphoreType.DMA,
    ],
    compiler_params=pltpu.CompilerParams(use_tc_tiling_on_sc=False),
)


def _mlp_bn_body(pA, pB, h, Wa, ba, Wb, bb, g, be, out):
  z = pA[...] + pB[...] - h[...]
  u = jnp.maximum(jnp.dot(z, Wa[...], preferred_element_type=jnp.float32)
                  + ba[...], 0.0)
  v = jnp.dot(u, Wb[...], preferred_element_type=jnp.float32) + bb[...]
  v = jnp.maximum(v, 0.0)
  m = jnp.mean(v, axis=0, keepdims=True)
  var = jnp.mean((v - m) * (v - m), axis=0, keepdims=True)
  out[...] = (v - m) * lax.rsqrt(var + 1e-5) * g[...] + be[...]


def _mlp_bn(parts, h, Wa, ba, Wb, bb, g, be, dout):
  return pl.pallas_call(
      _mlp_bn_body,
      out_shape=jax.ShapeDtypeStruct((N, dout), jnp.float32),
  )(parts[0], parts[1], h, Wa, ba, Wb, bb, g, be)


def kernel(x, edge_index, W1a, b1a, W1b, b1b, g1, be1,
           W2a, b2a, W2b, b2b, g2, be2):
  src2 = edge_index[0].reshape(NW * STEPS, CH)
  dst2 = edge_index[1].reshape(NW * STEPS, CH)
  parts1 = _agg(x, src2, dst2)
  h1 = _mlp_bn(parts1, x, W1a, b1a, W1b, b1b, g1, be1, F)
  parts2 = _agg(h1, src2, dst2)
  h2 = _mlp_bn(parts2, h1, W2a, b2a, W2b, b2b, g2, be2, 2)
  return h2
